# Initial kernel scaffold; baseline (speedup 1.0000x reference)
import jax
import jax.numpy as jnp
from jax.experimental import pallas as pl


def kernel(user_emb, item_emb, adj_values, mm_values, adj_indices, mm_indices):
    raise NotImplementedError("write your pallas kernel here")


# speedup vs baseline: 1.9240x; 1.9240x over previous
import functools

import jax
import jax.numpy as jnp
from jax import lax
from jax.experimental import pallas as pl
from jax.experimental.pallas import tpu as pltpu
from jax.experimental.pallas import tpu_sc as plsc

N_USERS = 25000
N_ITEMS = 25000
N_NODES = N_USERS + N_ITEMS
D = 64

SB = 128
E_ADJ_PAD = 802816
E_MM_PAD = 262144

WADJ = 1568
MM_SPLIT = 12800
WMM = 800

ACC_ROWS = 25600
ZBLK = 200
CBLK = 40

_MESH = plsc.VectorSubcoreMesh(core_axis_name="c", subcore_axis_name="s")


def _zero_buf(buf, nrows):
    def body(r, _):
        for k in range(D // 16):
            buf[r, pl.ds(k * 16, 16)] = jnp.zeros((16,), jnp.float32)
        return 0
    lax.fori_loop(0, nrows, body, 0)


def _zero_acc(acc, zbuf, s, blk):
    nz = ACC_ROWS // blk // 16
    for z in range(nz):
        pltpu.sync_copy(zbuf, acc.at[pl.ds((s * nz + z) * blk, blk)])


def _scan_compact(dst2d, nbig, lo, hi, pos_out, cnt_out, t,
                  dstb, poslist, cntb, epad):
    iota = lax.iota(jnp.int32, 16)

    def big(i, carry):
        cnt, nch = carry
        pltpu.sync_copy(dst2d.at[pl.ds(i * 4, 4)], dstb)
        for j in range(4):
            for g in range(8):
                d = dstb[j, pl.ds(g * 16, 16)]
                inb = (d >= lo) & (d < hi)
                pos16 = iota + ((i * 4 + j) * SB + g * 16)
                key = jnp.where(inb, iota, iota + 16)
                _, vs = plsc.sort_key_val(key, pos16)
                poslist[pl.ds(cnt, 16)] = vs
                cnt = cnt + plsc.all_reduce_population_count(inb)[0]
            nflush = cnt // SB

            @pl.when(nflush > 0)
            def _():
                pltpu.sync_copy(poslist.at[pl.ds(0, SB)],
                                pos_out.at[t, pl.ds(nch * SB, SB)])
                for g in range(8):
                    poslist[pl.ds(g * 16, 16)] = poslist[pl.ds(SB + g * 16, 16)]

            cnt = cnt - SB * nflush
            nch = nch + nflush
        return (cnt, nch)

    cnt, nch = lax.fori_loop(0, nbig, big, (jnp.int32(0), jnp.int32(0)))

    for g in range(8):
        sl = pl.ds(g * 16, 16)
        poslist[sl] = jnp.where(iota + g * 16 < cnt, poslist[sl], epad)

    @pl.when(cnt > 0)
    def _():
        pltpu.sync_copy(poslist.at[pl.ds(0, SB)],
                        pos_out.at[t, pl.ds(nch * SB, SB)])

    nch = nch + jnp.where(cnt > 0, 1, 0)
    for g in range(8):
        cntb[pl.ds(g * 16, 16)] = jnp.full((16,), nch, jnp.int32)
    pltpu.sync_copy(cntb, cnt_out.at[t])


@functools.partial(
    pl.kernel,
    out_type=(
        jax.ShapeDtypeStruct((32, E_ADJ_PAD), jnp.int32),
        jax.ShapeDtypeStruct((32, SB), jnp.int32),
        jax.ShapeDtypeStruct((32, E_MM_PAD), jnp.int32),
        jax.ShapeDtypeStruct((32, SB), jnp.int32),
    ),
    mesh=_MESH,
    scratch_types=[
        pltpu.VMEM((4, SB), jnp.int32),
        pltpu.VMEM((2 * SB,), jnp.int32),
        pltpu.VMEM((SB,), jnp.int32),
    ],
    compiler_params=pltpu.CompilerParams(
        use_tc_tiling_on_sc=False, needs_layout_passes=False),
)
def _prep(adj_dst2d, mm_dst2d,
          adj_pos, adj_cnt, mm_pos, mm_cnt,
          dstb, poslist, cntb):
    c = lax.axis_index("c")
    s = lax.axis_index("s")
    t = c * 16 + s

    lo = c * N_USERS + s * WADJ
    win = jnp.minimum(WADJ, N_USERS - s * WADJ)
    _scan_compact(adj_dst2d, E_ADJ_PAD // 512, lo, lo + win,
                  adj_pos, adj_cnt, t, dstb, poslist, cntb, E_ADJ_PAD - 1)

    half = jnp.where(c == 0, MM_SPLIT, N_ITEMS - MM_SPLIT)
    lo_m = c * MM_SPLIT + s * WMM
    win_m = jnp.minimum(WMM, half - s * WMM)
    _scan_compact(mm_dst2d, E_MM_PAD // 512, lo_m, lo_m + win_m,
                  mm_pos, mm_cnt, t, dstb, poslist, cntb, E_MM_PAD - 1)


def _spmm_pos(x_hbm, dflat, sflat, vflat, pos, cnt, acc,
              posb, dstw, srcw, valw, sidxb, rowsb, cntb, sem,
              t, base, wlo, win):
    pltpu.sync_copy(cnt.at[t], cntb)
    nch = cntb[pl.ds(0, 16)][0]

    def chunk(ch, _):
        pltpu.sync_copy(pos.at[t, pl.ds(ch * SB, SB)], posb)
        cp1 = pltpu.async_copy(dflat.at[posb], dstw, sem)
        cp2 = pltpu.async_copy(sflat.at[posb], srcw, sem)
        cp3 = pltpu.async_copy(vflat.at[posb], valw, sem)
        cp1.wait(); cp2.wait(); cp3.wait()
        for g in range(8):
            sl = pl.ds(g * 16, 16)
            rel = dstw[sl] - base
            sidxb[0, sl] = jnp.minimum(jnp.maximum(rel, wlo), wlo + win - 1)
        pltpu.async_copy(x_hbm.at[srcw], rowsb, sem).wait()

        @plsc.parallel_loop(0, SB // 16)
        def scale(g):
            vv = valw[pl.ds(g * 16, 16)]
            for lane in range(16):
                v = vv[lane]
                e = g * 16 + lane
                for k in range(D // 16):
                    sl = pl.ds(k * 16, 16)
                    rowsb[e, sl] = rowsb[e, sl] * v

        pltpu.sync_copy(rowsb, acc.at[sidxb.at[0]], add=True)
        return 0

    lax.fori_loop(0, nch, chunk, 0)


_SCRATCH_SPMM = [
    pltpu.VMEM_SHARED((ACC_ROWS, D), jnp.float32),
    pltpu.VMEM((SB,), jnp.int32),
    pltpu.VMEM((SB,), jnp.int32),
    pltpu.VMEM((SB,), jnp.int32),
    pltpu.VMEM((SB,), jnp.float32),
    pltpu.VMEM((1, SB), jnp.int32),
    pltpu.VMEM((SB, D), jnp.float32),
    pltpu.VMEM((SB,), jnp.int32),
    pltpu.SemaphoreType.DMA,
]


@functools.partial(
    pl.kernel,
    out_type=(
        jax.ShapeDtypeStruct((N_ITEMS, D), jnp.float32),
        jax.ShapeDtypeStruct((N_NODES, D), jnp.float32),
    ),
    mesh=_MESH,
    scratch_types=_SCRATCH_SPMM + [
        pltpu.VMEM((ZBLK, D), jnp.float32),
    ],
    compiler_params=pltpu.CompilerParams(use_tc_tiling_on_sc=False),
)
def _phase_a(ego0, adj_d, adj_s, adj_v, mm_d, mm_s, mm_v,
             adj_pos, adj_cnt, mm_pos, mm_cnt,
             h_out, ego1_out,
             acc, posb, dstw, srcw, valw, sidxb, rowsb, cntb, sem, a2b):
    c = lax.axis_index("c")
    s = lax.axis_index("s")
    t = c * 16 + s

    _zero_buf(a2b, ZBLK)
    _zero_acc(acc, a2b, s, ZBLK)
    plsc.subcore_barrier()

    half = jnp.where(c == 0, MM_SPLIT, N_ITEMS - MM_SPLIT)
    win_m = jnp.minimum(WMM, half - s * WMM)
    _spmm_pos(ego0, mm_d, mm_s, mm_v, mm_pos, mm_cnt, acc,
              posb, dstw, srcw, valw, sidxb, rowsb, cntb, sem,
              t, c * MM_SPLIT, s * WMM, win_m)
    plsc.subcore_barrier()

    nblk = jnp.where(c == 0, MM_SPLIT // ZBLK, (N_ITEMS - MM_SPLIT) // ZBLK)

    def hblk(ib, _):
        r0 = (s + ib * 16) * ZBLK
        pltpu.sync_copy(acc.at[pl.ds(r0, ZBLK)], a2b)
        pltpu.sync_copy(a2b, h_out.at[pl.ds(c * MM_SPLIT + r0, ZBLK)])
        return 0

    lax.fori_loop(0, (nblk - s + 15) // 16, hblk, 0)
    plsc.subcore_barrier()

    _zero_buf(a2b, ZBLK)
    _zero_acc(acc, a2b, s, ZBLK)
    plsc.subcore_barrier()

    win = jnp.minimum(WADJ, N_USERS - s * WADJ)
    _spmm_pos(ego0, adj_d, adj_s, adj_v, adj_pos, adj_cnt, acc,
              posb, dstw, srcw, valw, sidxb, rowsb, cntb, sem,
              t, c * N_USERS, s * WADJ, win)
    plsc.subcore_barrier()

    def eblk(ib, _):
        r0 = (s + ib * 16) * ZBLK
        pltpu.sync_copy(acc.at[pl.ds(r0, ZBLK)], a2b)
        pltpu.sync_copy(a2b, ego1_out.at[pl.ds(c * N_USERS + r0, ZBLK)])
        return 0

    lax.fori_loop(0, (125 - s + 15) // 16, eblk, 0)


@functools.partial(
    pl.kernel,
    out_type=jax.ShapeDtypeStruct((N_NODES, D), jnp.float32),
    mesh=_MESH,
    scratch_types=_SCRATCH_SPMM + [
        pltpu.VMEM((CBLK, D), jnp.float32),
        pltpu.VMEM((CBLK, D), jnp.float32),
        pltpu.VMEM((CBLK, D), jnp.float32),
        pltpu.VMEM((CBLK, D), jnp.float32),
    ],
    compiler_params=pltpu.CompilerParams(use_tc_tiling_on_sc=False),
)
def _phase_b(ego0, ego1, h, adj_d, adj_s, adj_v, adj_pos, adj_cnt,
             out,
             acc, posb, dstw, srcw, valw, sidxb, rowsb, cntb, sem,
             a0b, a1b, a2b, hbuf):
    c = lax.axis_index("c")
    s = lax.axis_index("s")
    t = c * 16 + s

    _zero_buf(a0b, CBLK)
    _zero_acc(acc, a0b, s, CBLK)
    plsc.subcore_barrier()

    win = jnp.minimum(WADJ, N_USERS - s * WADJ)
    _spmm_pos(ego1, adj_d, adj_s, adj_v, adj_pos, adj_cnt, acc,
              posb, dstw, srcw, valw, sidxb, rowsb, cntb, sem,
              t, c * N_USERS, s * WADJ, win)
    plsc.subcore_barrier()

    cf = jnp.where(c == 1, 1.0, 0.0).astype(jnp.float32)
    nblk = N_USERS // CBLK

    def cblk(ib, _):
        r0 = (s + ib * 16) * CBLK
        g0 = c * N_USERS + r0
        pltpu.sync_copy(acc.at[pl.ds(r0, CBLK)], a2b)
        pltpu.sync_copy(ego0.at[pl.ds(g0, CBLK)], a0b)
        pltpu.sync_copy(ego1.at[pl.ds(g0, CBLK)], a1b)
        pltpu.sync_copy(h.at[pl.ds(r0, CBLK)], hbuf)

        @plsc.parallel_loop(0, CBLK, unroll=2)
        def comb(r):
            for k in range(D // 16):
                sl = pl.ds(k * 16, 16)
                m = (a0b[r, sl] + a1b[r, sl] + a2b[r, sl]) * (1.0 / 3.0)
                a0b[r, sl] = m + hbuf[r, sl] * cf

        pltpu.sync_copy(a0b, out.at[pl.ds(g0, CBLK)])
        return 0

    lax.fori_loop(0, (nblk - s + 15) // 16, cblk, 0)


def _pad1d(x, n, fill):
    pad = jnp.full((n - x.shape[0],), fill, x.dtype)
    return jnp.concatenate([x, pad])


def kernel(user_emb, item_emb, adj_values, mm_values, adj_indices, mm_indices):
    ego0 = jnp.concatenate([user_emb, item_emb], axis=0)
    adj_d = _pad1d(adj_indices[0], E_ADJ_PAD, 0)
    adj_s = _pad1d(adj_indices[1], E_ADJ_PAD, 0)
    adj_v = _pad1d(adj_values, E_ADJ_PAD, 0.0)
    mm_d = _pad1d(mm_indices[0], E_MM_PAD, 0)
    mm_s = _pad1d(mm_indices[1] + N_USERS, E_MM_PAD, 0)
    mm_v = _pad1d(mm_values, E_MM_PAD, 0.0)

    adj_pos, adj_cnt, mm_pos, mm_cnt = _prep(
        adj_d.reshape(-1, SB), mm_d.reshape(-1, SB))
    h, ego1 = _phase_a(ego0, adj_d, adj_s, adj_v, mm_d, mm_s, mm_v,
                       adj_pos, adj_cnt, mm_pos, mm_cnt)
    out = _phase_b(ego0, ego1, h, adj_d, adj_s, adj_v, adj_pos, adj_cnt)
    return out


# speedup vs baseline: 1.9943x; 1.0365x over previous
import functools

import jax
import jax.numpy as jnp
from jax import lax
from jax.experimental import pallas as pl
from jax.experimental.pallas import tpu as pltpu
from jax.experimental.pallas import tpu_sc as plsc

N_USERS = 25000
N_ITEMS = 25000
N_NODES = N_USERS + N_ITEMS
D = 64

SB = 128
E_ADJ_PAD = 802816
E_MM_PAD = 262144

WADJ = 1568
MM_SPLIT = 12800
WMM = 800

ACC_ROWS = 25600
ZBLK = 200
CBLK = 40

_MESH = plsc.VectorSubcoreMesh(core_axis_name="c", subcore_axis_name="s")


def _zero_buf(buf, nrows):
    def body(r, _):
        for k in range(D // 16):
            buf[r, pl.ds(k * 16, 16)] = jnp.zeros((16,), jnp.float32)
        return 0
    lax.fori_loop(0, nrows, body, 0)


def _zero_acc(acc, zbuf, s, blk):
    nz = ACC_ROWS // blk // 16
    for z in range(nz):
        pltpu.sync_copy(zbuf, acc.at[pl.ds((s * nz + z) * blk, blk)])


def _scan_compact(dst2d, nbig, lo, hi, pos_out, cnt_out, t,
                  dstb, poslist, cntb, epad):
    iota = lax.iota(jnp.int32, 16)

    def big(i, carry):
        cnt = list(carry[0:4])
        nch = list(carry[4:8])
        pltpu.sync_copy(dst2d.at[pl.ds(i * 4, 4)], dstb)
        for j in range(4):
            for g in range(8):
                l = g % 4
                d = dstb[j, pl.ds(g * 16, 16)]
                inb = (d >= lo) & (d < hi)
                pos16 = iota + ((i * 4 + j) * SB + g * 16)
                key = jnp.where(inb, iota, iota + 16)
                _, vs = plsc.sort_key_val(key, pos16)
                poslist[l, pl.ds(cnt[l], 16)] = vs
                cnt[l] = cnt[l] + plsc.all_reduce_population_count(inb)[0]
            for l in range(4):
                nflush = cnt[l] // SB

                @pl.when(nflush > 0)
                def _(l=l):
                    pltpu.sync_copy(poslist.at[l, pl.ds(0, SB)],
                                    pos_out.at[t, l, pl.ds(nch[l] * SB, SB)])
                    for g in range(8):
                        poslist[l, pl.ds(g * 16, 16)] = (
                            poslist[l, pl.ds(SB + g * 16, 16)])

                cnt[l] = cnt[l] - SB * nflush
                nch[l] = nch[l] + nflush
        return tuple(cnt) + tuple(nch)

    z = jnp.int32(0)
    carry = lax.fori_loop(0, nbig, big, (z, z, z, z, z, z, z, z))
    cnt = list(carry[0:4])
    nch = list(carry[4:8])

    for l in range(4):
        for g in range(8):
            sl = pl.ds(g * 16, 16)
            poslist[l, sl] = jnp.where(iota + g * 16 < cnt[l],
                                       poslist[l, sl], epad)

        @pl.when(cnt[l] > 0)
        def _(l=l):
            pltpu.sync_copy(poslist.at[l, pl.ds(0, SB)],
                            pos_out.at[t, l, pl.ds(nch[l] * SB, SB)])

        nch[l] = nch[l] + jnp.where(cnt[l] > 0, 1, 0)

    cv = jnp.full((16,), 0, jnp.int32)
    for l in range(4):
        cv = jnp.where(iota == l, nch[l], cv)
    for g in range(8):
        cntb[pl.ds(g * 16, 16)] = cv
    pltpu.sync_copy(cntb, cnt_out.at[t])


@functools.partial(
    pl.kernel,
    out_type=(
        jax.ShapeDtypeStruct((32, 4, E_ADJ_PAD // 4 + SB), jnp.int32),
        jax.ShapeDtypeStruct((32, SB), jnp.int32),
        jax.ShapeDtypeStruct((32, 4, E_MM_PAD // 4 + SB), jnp.int32),
        jax.ShapeDtypeStruct((32, SB), jnp.int32),
    ),
    mesh=_MESH,
    scratch_types=[
        pltpu.VMEM((4, SB), jnp.int32),
        pltpu.VMEM((4, 2 * SB), jnp.int32),
        pltpu.VMEM((SB,), jnp.int32),
    ],
    compiler_params=pltpu.CompilerParams(
        use_tc_tiling_on_sc=False, needs_layout_passes=False),
)
def _prep(adj_dst2d, mm_dst2d,
          adj_pos, adj_cnt, mm_pos, mm_cnt,
          dstb, poslist, cntb):
    c = lax.axis_index("c")
    s = lax.axis_index("s")
    t = c * 16 + s

    lo = c * N_USERS + s * WADJ
    win = jnp.minimum(WADJ, N_USERS - s * WADJ)
    _scan_compact(adj_dst2d, E_ADJ_PAD // 512, lo, lo + win,
                  adj_pos, adj_cnt, t, dstb, poslist, cntb, E_ADJ_PAD - 1)

    half = jnp.where(c == 0, MM_SPLIT, N_ITEMS - MM_SPLIT)
    lo_m = c * MM_SPLIT + s * WMM
    win_m = jnp.minimum(WMM, half - s * WMM)
    _scan_compact(mm_dst2d, E_MM_PAD // 512, lo_m, lo_m + win_m,
                  mm_pos, mm_cnt, t, dstb, poslist, cntb, E_MM_PAD - 1)


def _spmm_pos(x_hbm, dflat, sflat, vflat, pos, cnt, acc,
              posb, dstw, srcw, valw, sidxb, rowsb, cntb, sem,
              t, base, wlo, win):
    pltpu.sync_copy(cnt.at[t], cntb)
    cv = cntb[pl.ds(0, 16)]

    def chunk(lch, _):
        l = lch // 65536
        ch = lch % 65536
        pltpu.sync_copy(pos.at[t, l, pl.ds(ch * SB, SB)], posb)
        cp1 = pltpu.async_copy(dflat.at[posb], dstw, sem)
        cp2 = pltpu.async_copy(sflat.at[posb], srcw, sem)
        cp3 = pltpu.async_copy(vflat.at[posb], valw, sem)
        cp1.wait(); cp2.wait(); cp3.wait()
        for g in range(8):
            sl = pl.ds(g * 16, 16)
            rel = dstw[sl] - base
            sidxb[0, sl] = jnp.minimum(jnp.maximum(rel, wlo), wlo + win - 1)
        pltpu.async_copy(x_hbm.at[srcw], rowsb, sem).wait()

        @plsc.parallel_loop(0, SB // 16)
        def scale(g):
            vv = valw[pl.ds(g * 16, 16)]
            for lane in range(16):
                v = vv[lane]
                e = g * 16 + lane
                for k in range(D // 16):
                    sl = pl.ds(k * 16, 16)
                    rowsb[e, sl] = rowsb[e, sl] * v

        pltpu.sync_copy(rowsb, acc.at[sidxb.at[0]], add=True)
        return 0

    for l in range(4):
        nch = cv[l]
        lax.fori_loop(l * 65536, l * 65536 + nch, chunk, 0)


_SCRATCH_SPMM = [
    pltpu.VMEM_SHARED((ACC_ROWS, D), jnp.float32),
    pltpu.VMEM((SB,), jnp.int32),
    pltpu.VMEM((SB,), jnp.int32),
    pltpu.VMEM((SB,), jnp.int32),
    pltpu.VMEM((SB,), jnp.float32),
    pltpu.VMEM((1, SB), jnp.int32),
    pltpu.VMEM((SB, D), jnp.float32),
    pltpu.VMEM((SB,), jnp.int32),
    pltpu.SemaphoreType.DMA,
]


@functools.partial(
    pl.kernel,
    out_type=(
        jax.ShapeDtypeStruct((N_ITEMS, D), jnp.float32),
        jax.ShapeDtypeStruct((N_NODES, D), jnp.float32),
    ),
    mesh=_MESH,
    scratch_types=_SCRATCH_SPMM + [
        pltpu.VMEM((ZBLK, D), jnp.float32),
    ],
    compiler_params=pltpu.CompilerParams(use_tc_tiling_on_sc=False),
)
def _phase_a(ego0, adj_d, adj_s, adj_v, mm_d, mm_s, mm_v,
             adj_pos, adj_cnt, mm_pos, mm_cnt,
             h_out, ego1_out,
             acc, posb, dstw, srcw, valw, sidxb, rowsb, cntb, sem, a2b):
    c = lax.axis_index("c")
    s = lax.axis_index("s")
    t = c * 16 + s

    _zero_buf(a2b, ZBLK)
    _zero_acc(acc, a2b, s, ZBLK)
    plsc.subcore_barrier()

    half = jnp.where(c == 0, MM_SPLIT, N_ITEMS - MM_SPLIT)
    win_m = jnp.minimum(WMM, half - s * WMM)
    _spmm_pos(ego0, mm_d, mm_s, mm_v, mm_pos, mm_cnt, acc,
              posb, dstw, srcw, valw, sidxb, rowsb, cntb, sem,
              t, c * MM_SPLIT, s * WMM, win_m)
    plsc.subcore_barrier()

    nblk = jnp.where(c == 0, MM_SPLIT // ZBLK, (N_ITEMS - MM_SPLIT) // ZBLK)

    def hblk(ib, _):
        r0 = (s + ib * 16) * ZBLK
        pltpu.sync_copy(acc.at[pl.ds(r0, ZBLK)], a2b)
        pltpu.sync_copy(a2b, h_out.at[pl.ds(c * MM_SPLIT + r0, ZBLK)])
        return 0

    lax.fori_loop(0, (nblk - s + 15) // 16, hblk, 0)
    plsc.subcore_barrier()

    _zero_buf(a2b, ZBLK)
    _zero_acc(acc, a2b, s, ZBLK)
    plsc.subcore_barrier()

    win = jnp.minimum(WADJ, N_USERS - s * WADJ)
    _spmm_pos(ego0, adj_d, adj_s, adj_v, adj_pos, adj_cnt, acc,
              posb, dstw, srcw, valw, sidxb, rowsb, cntb, sem,
              t, c * N_USERS, s * WADJ, win)
    plsc.subcore_barrier()

    def eblk(ib, _):
        r0 = (s + ib * 16) * ZBLK
        pltpu.sync_copy(acc.at[pl.ds(r0, ZBLK)], a2b)
        pltpu.sync_copy(a2b, ego1_out.at[pl.ds(c * N_USERS + r0, ZBLK)])
        return 0

    lax.fori_loop(0, (125 - s + 15) // 16, eblk, 0)


@functools.partial(
    pl.kernel,
    out_type=jax.ShapeDtypeStruct((N_NODES, D), jnp.float32),
    mesh=_MESH,
    scratch_types=_SCRATCH_SPMM + [
        pltpu.VMEM((CBLK, D), jnp.float32),
        pltpu.VMEM((CBLK, D), jnp.float32),
        pltpu.VMEM((CBLK, D), jnp.float32),
        pltpu.VMEM((CBLK, D), jnp.float32),
    ],
    compiler_params=pltpu.CompilerParams(use_tc_tiling_on_sc=False),
)
def _phase_b(ego0, ego1, h, adj_d, adj_s, adj_v, adj_pos, adj_cnt,
             out,
             acc, posb, dstw, srcw, valw, sidxb, rowsb, cntb, sem,
             a0b, a1b, a2b, hbuf):
    c = lax.axis_index("c")
    s = lax.axis_index("s")
    t = c * 16 + s

    _zero_buf(a0b, CBLK)
    _zero_acc(acc, a0b, s, CBLK)
    plsc.subcore_barrier()

    win = jnp.minimum(WADJ, N_USERS - s * WADJ)
    _spmm_pos(ego1, adj_d, adj_s, adj_v, adj_pos, adj_cnt, acc,
              posb, dstw, srcw, valw, sidxb, rowsb, cntb, sem,
              t, c * N_USERS, s * WADJ, win)
    plsc.subcore_barrier()

    cf = jnp.where(c == 1, 1.0, 0.0).astype(jnp.float32)
    nblk = N_USERS // CBLK

    def cblk(ib, _):
        r0 = (s + ib * 16) * CBLK
        g0 = c * N_USERS + r0
        pltpu.sync_copy(acc.at[pl.ds(r0, CBLK)], a2b)
        pltpu.sync_copy(ego0.at[pl.ds(g0, CBLK)], a0b)
        pltpu.sync_copy(ego1.at[pl.ds(g0, CBLK)], a1b)
        pltpu.sync_copy(h.at[pl.ds(r0, CBLK)], hbuf)

        @plsc.parallel_loop(0, CBLK, unroll=2)
        def comb(r):
            for k in range(D // 16):
                sl = pl.ds(k * 16, 16)
                m = (a0b[r, sl] + a1b[r, sl] + a2b[r, sl]) * (1.0 / 3.0)
                a0b[r, sl] = m + hbuf[r, sl] * cf

        pltpu.sync_copy(a0b, out.at[pl.ds(g0, CBLK)])
        return 0

    lax.fori_loop(0, (nblk - s + 15) // 16, cblk, 0)


def _pad1d(x, n, fill):
    pad = jnp.full((n - x.shape[0],), fill, x.dtype)
    return jnp.concatenate([x, pad])


def _pad_dst(x, n, starts):
    npad = n - x.shape[0]
    pad = jnp.asarray(starts, x.dtype)[jnp.arange(npad) % 32]
    return jnp.concatenate([x, pad])


def kernel(user_emb, item_emb, adj_values, mm_values, adj_indices, mm_indices):
    ego0 = jnp.concatenate([user_emb, item_emb], axis=0)
    adj_starts = [(w // 16) * N_USERS + (w % 16) * WADJ for w in range(32)]
    mm_starts = [(w // 16) * MM_SPLIT + (w % 16) * WMM for w in range(32)]
    adj_d = _pad_dst(adj_indices[0], E_ADJ_PAD, adj_starts)
    adj_s = _pad1d(adj_indices[1], E_ADJ_PAD, 0)
    adj_v = _pad1d(adj_values, E_ADJ_PAD, 0.0)
    mm_d = _pad_dst(mm_indices[0], E_MM_PAD, mm_starts)
    mm_s = _pad1d(mm_indices[1] + N_USERS, E_MM_PAD, 0)
    mm_v = _pad1d(mm_values, E_MM_PAD, 0.0)

    adj_pos, adj_cnt, mm_pos, mm_cnt = _prep(
        adj_d.reshape(-1, SB), mm_d.reshape(-1, SB))
    h, ego1 = _phase_a(ego0, adj_d, adj_s, adj_v, mm_d, mm_s, mm_v,
                       adj_pos, adj_cnt, mm_pos, mm_cnt)
    out = _phase_b(ego0, ego1, h, adj_d, adj_s, adj_v, adj_pos, adj_cnt)
    return out
